# baseline (speedup 1.0000x reference)
import functools
import jax
import jax.numpy as jnp
from jax import lax
from jax.experimental import pallas as pl
from jax.experimental.pallas import tpu as pltpu
from jax.experimental.pallas import tpu_sc as plsc

_B, _N, _IN = 8, 2048, 3
_K = 16
_F = 254
_FP = 256
_R = 256
_NW = 32
_BC = 1
_PPW = (_BC * _N) // _NW
_G = 8
_GROUPS = _PPW // _G


def _mlp_body(x_ref, w1_ref, b1_ref, w2_ref, b2_ref, o_ref):
    xb = x_ref[...]
    h = jnp.dot(xb, w1_ref[...], preferred_element_type=jnp.float32)
    h = h + b1_ref[...]
    h = 0.5 * h * (1.0 + lax.erf(h * 0.7071067811865476))
    r = jnp.dot(h, w2_ref[...], preferred_element_type=jnp.float32)
    o_ref[...] = r + b2_ref[...]


_CW = 128
_NCH = _N // _CW
_SB = 32


def _topk_body(xt_ref, x_ref, o_ref, dist_ref):
    b = pl.program_id(0)
    xa = xt_ref[0]
    xb = x_ref[0]
    dots = jnp.dot(xb, xa, preferred_element_type=jnp.float32)
    sq_a = jnp.sum(xa * xa, axis=0, keepdims=True)
    sq_b = jnp.sum(xb * xb, axis=1, keepdims=True)
    dist_ref[...] = sq_b + sq_a - 2.0 * dots
    iota_f = lax.broadcasted_iota(jnp.int32, (_R, _N), 1).astype(jnp.float32)
    base = b * _N
    mprev = jnp.full((_R, 1), -jnp.inf, jnp.float32)
    iprev = jnp.full((_R, 1), -1.0, jnp.float32)
    for k in range(_K):
        d = dist_ref[...]
        sel = (d > mprev) | ((d == mprev) & (iota_f > iprev))
        dm = jnp.where(sel, d, jnp.inf)
        m = jnp.min(dm, axis=1, keepdims=True)
        idxf = jnp.min(jnp.where(dm == m, iota_f, jnp.float32(2 * _N)),
                       axis=1, keepdims=True)
        o_ref[0, :, k] = idxf[:, 0].astype(jnp.int32) + base
        mprev, iprev = m, idxf


def _sc_pool_body(res_hbm, idx_hbm, out_hbm, idx0_v, idx1_v, rows0_v, rows1_v,
                  outb_v, sem0, sem1):
    wid = lax.axis_index("s") * 2 + lax.axis_index("c")
    base_pt = wid * _PPW

    def start(g, idxb, rowsb, sem):
        pltpu.sync_copy(
            idx_hbm.at[pl.ds((base_pt + g * _G) * _K, _G * _K)], idxb)
        pltpu.async_copy(res_hbm.at[idxb], rowsb, sem)

    def wait(rowsb, sem):
        pltpu.make_async_copy(res_hbm.at[pl.ds(0, _G * _K)], rowsb, sem).wait()

    def compute(g, rowsb):
        def point(pt, _):
            r0 = pt * _K
            for fc in range(_FP // 16):
                v = [rowsb[r0 + nb, pl.ds(fc * 16, 16)] for nb in range(_K)]
                while len(v) > 1:
                    v = [jnp.maximum(v[2 * i], v[2 * i + 1])
                         for i in range(len(v) // 2)]
                outb_v[pt, pl.ds(fc * 16, 16)] = v[0]
            return 0

        lax.fori_loop(0, _G, point, 0)
        pltpu.sync_copy(outb_v, out_hbm.at[pl.ds(base_pt + g * _G, _G)])

    start(0, idx0_v, rows0_v, sem0)

    def pair(p, _):
        g0 = 2 * p
        start(g0 + 1, idx1_v, rows1_v, sem1)
        wait(rows0_v, sem0)
        compute(g0, rows0_v)
        start(jnp.minimum(g0 + 2, _GROUPS - 1), idx0_v, rows0_v, sem0)
        wait(rows1_v, sem1)
        compute(g0 + 1, rows1_v)
        return 0

    lax.fori_loop(0, _GROUPS // 2, pair, 0)
    wait(rows0_v, sem0)


def _finalize_body(p_ref, x_ref, o_ref):
    pm = p_ref[0][:, :_F]
    gv = jnp.max(pm, axis=0, keepdims=True)
    o_ref[0] = jnp.concatenate(
        [pm, jnp.broadcast_to(gv, (_N, _F)), x_ref[0]], axis=1)


def _chunk_pipeline(xh, w1t, b1r, w2t, b2r):
    x2d = xh.reshape(_BC * _N, _IN)
    res = pl.pallas_call(
        _mlp_body,
        grid=(_BC * _N // 512,),
        in_specs=[
            pl.BlockSpec((512, _IN), lambda i: (i, 0)),
            pl.BlockSpec((_IN, _F), lambda i: (0, 0)),
            pl.BlockSpec((1, _F), lambda i: (0, 0)),
            pl.BlockSpec((_F, _FP), lambda i: (0, 0)),
            pl.BlockSpec((1, _FP), lambda i: (0, 0)),
        ],
        out_specs=pl.BlockSpec((512, _FP), lambda i: (i, 0)),
        out_shape=jax.ShapeDtypeStruct((_BC * _N, _FP), jnp.float32),
    )(x2d, w1t, b1r, w2t, b2r)

    xt = jnp.transpose(xh, (0, 2, 1))
    knn = pl.pallas_call(
        _topk_body,
        grid=(_BC, _N // _R),
        in_specs=[
            pl.BlockSpec((1, _IN, _N), lambda b, i: (b, 0, 0)),
            pl.BlockSpec((1, _R, _IN), lambda b, i: (b, i, 0)),
        ],
        out_specs=pl.BlockSpec((1, _R, _K), lambda b, i: (b, i, 0)),
        out_shape=jax.ShapeDtypeStruct((_BC, _N, _K), jnp.int32),
        scratch_shapes=[pltpu.VMEM((_R, _N), jnp.float32)],
    )(xt, xh)

    idx_flat = knn.reshape(_BC * _N * _K)

    mesh = plsc.VectorSubcoreMesh(core_axis_name="c", subcore_axis_name="s")
    pooled = pl.kernel(
        _sc_pool_body,
        mesh=mesh,
        out_type=jax.ShapeDtypeStruct((_BC * _N, _FP), jnp.float32),
        scratch_types=[
            pltpu.VMEM((_G * _K,), jnp.int32),
            pltpu.VMEM((_G * _K,), jnp.int32),
            pltpu.VMEM((_G * _K, _FP), jnp.float32),
            pltpu.VMEM((_G * _K, _FP), jnp.float32),
            pltpu.VMEM((_G, _FP), jnp.float32),
            pltpu.SemaphoreType.DMA,
            pltpu.SemaphoreType.DMA,
        ],
    )(res, idx_flat)
    return pooled


def kernel(x, W1, b1, W2, b2):
    w1t = W1.T
    w2t = jnp.pad(W2, ((0, _FP - _F), (0, 0))).T
    b1r = b1.reshape(1, _F)
    b2r = jnp.pad(b2, (0, _FP - _F)).reshape(1, _FP)

    pooled = [_chunk_pipeline(x[c:c + _BC], w1t, b1r, w2t, b2r)
              for c in range(0, _B, _BC)]

    pooled3 = jnp.concatenate(pooled).reshape(_B, _N, _FP)
    out = pl.pallas_call(
        _finalize_body,
        grid=(_B,),
        in_specs=[
            pl.BlockSpec((1, _N, _FP), lambda b: (b, 0, 0)),
            pl.BlockSpec((1, _N, _IN), lambda b: (b, 0, 0)),
        ],
        out_specs=pl.BlockSpec((1, _N, 2 * _F + _IN), lambda b: (b, 0, 0)),
        out_shape=jax.ShapeDtypeStruct((_B, _N, 2 * _F + _IN), jnp.float32),
    )(pooled3, x)
    return out


# speedup vs baseline: 1.4932x; 1.4932x over previous
import functools
import jax
import jax.numpy as jnp
from jax import lax
from jax.experimental import pallas as pl
from jax.experimental.pallas import tpu as pltpu
from jax.experimental.pallas import tpu_sc as plsc

_B, _N, _IN = 8, 2048, 3
_K = 16
_F = 254
_FP = 256
_R = 256
_NW = 32
_BC = 1
_PPW = (_BC * _N) // _NW
_G = 8
_GROUPS = _PPW // _G


def _mlp_body(x_ref, w1_ref, b1_ref, w2_ref, b2_ref, o_ref):
    xb = x_ref[...]
    h = jnp.dot(xb, w1_ref[...], preferred_element_type=jnp.float32)
    h = h + b1_ref[...]
    h = 0.5 * h * (1.0 + lax.erf(h * 0.7071067811865476))
    r = jnp.dot(h, w2_ref[...], preferred_element_type=jnp.float32)
    o_ref[...] = r + b2_ref[...]


_CW = 128
_NCH = _N // _CW
_SB = 32


def _topk_body(xt_ref, x_ref, o_ref, dist_ref):
    b = pl.program_id(0)
    xa = xt_ref[0]
    xb = x_ref[0]
    dots = jnp.dot(xb, xa, preferred_element_type=jnp.float32)
    sq_a = jnp.sum(xa * xa, axis=0, keepdims=True)
    sq_b = jnp.sum(xb * xb, axis=1, keepdims=True)
    dist_ref[...] = sq_b + sq_a - 2.0 * dots
    iota_f = lax.broadcasted_iota(jnp.int32, (_R, _N), 1).astype(jnp.float32)
    base = b * _N
    for k in range(_K):
        d = dist_ref[...]
        m = jnp.min(d, axis=1, keepdims=True)
        eq = d == m
        idxf = jnp.min(jnp.where(eq, iota_f, jnp.float32(2 * _N)), axis=1)
        o_ref[0, :, k] = idxf.astype(jnp.int32) + base
        dist_ref[...] = jnp.where(iota_f == idxf[:, None], jnp.inf, d)


def _sc_pool_body(res_hbm, idx_hbm, out_hbm, idx0_v, idx1_v, rows0_v, rows1_v,
                  outb_v, sem0, sem1):
    wid = lax.axis_index("s") * 2 + lax.axis_index("c")
    base_pt = wid * _PPW

    def start(g, idxb, rowsb, sem):
        pltpu.sync_copy(
            idx_hbm.at[pl.ds((base_pt + g * _G) * _K, _G * _K)], idxb)
        pltpu.async_copy(res_hbm.at[idxb], rowsb, sem)

    def wait(rowsb, sem):
        pltpu.make_async_copy(res_hbm.at[pl.ds(0, _G * _K)], rowsb, sem).wait()

    def compute(g, rowsb):
        def point(pt, _):
            r0 = pt * _K
            for fc in range(_FP // 16):
                v = [rowsb[r0 + nb, pl.ds(fc * 16, 16)] for nb in range(_K)]
                while len(v) > 1:
                    v = [jnp.maximum(v[2 * i], v[2 * i + 1])
                         for i in range(len(v) // 2)]
                outb_v[pt, pl.ds(fc * 16, 16)] = v[0]
            return 0

        lax.fori_loop(0, _G, point, 0)
        pltpu.sync_copy(outb_v, out_hbm.at[pl.ds(base_pt + g * _G, _G)])

    start(0, idx0_v, rows0_v, sem0)

    def pair(p, _):
        g0 = 2 * p
        start(g0 + 1, idx1_v, rows1_v, sem1)
        wait(rows0_v, sem0)
        compute(g0, rows0_v)
        start(jnp.minimum(g0 + 2, _GROUPS - 1), idx0_v, rows0_v, sem0)
        wait(rows1_v, sem1)
        compute(g0 + 1, rows1_v)
        return 0

    lax.fori_loop(0, _GROUPS // 2, pair, 0)
    wait(rows0_v, sem0)


def _finalize_body(p_ref, x_ref, o_ref):
    pm = p_ref[0][:, :_F]
    gv = jnp.max(pm, axis=0, keepdims=True)
    o_ref[0] = jnp.concatenate(
        [pm, jnp.broadcast_to(gv, (_N, _F)), x_ref[0]], axis=1)


def _chunk_pipeline(xh, w1t, b1r, w2t, b2r):
    x2d = xh.reshape(_BC * _N, _IN)
    res = pl.pallas_call(
        _mlp_body,
        grid=(_BC * _N // 512,),
        in_specs=[
            pl.BlockSpec((512, _IN), lambda i: (i, 0)),
            pl.BlockSpec((_IN, _F), lambda i: (0, 0)),
            pl.BlockSpec((1, _F), lambda i: (0, 0)),
            pl.BlockSpec((_F, _FP), lambda i: (0, 0)),
            pl.BlockSpec((1, _FP), lambda i: (0, 0)),
        ],
        out_specs=pl.BlockSpec((512, _FP), lambda i: (i, 0)),
        out_shape=jax.ShapeDtypeStruct((_BC * _N, _FP), jnp.float32),
    )(x2d, w1t, b1r, w2t, b2r)

    xt = jnp.transpose(xh, (0, 2, 1))
    knn = pl.pallas_call(
        _topk_body,
        grid=(_BC, _N // _R),
        in_specs=[
            pl.BlockSpec((1, _IN, _N), lambda b, i: (b, 0, 0)),
            pl.BlockSpec((1, _R, _IN), lambda b, i: (b, i, 0)),
        ],
        out_specs=pl.BlockSpec((1, _R, _K), lambda b, i: (b, i, 0)),
        out_shape=jax.ShapeDtypeStruct((_BC, _N, _K), jnp.int32),
        scratch_shapes=[pltpu.VMEM((_R, _N), jnp.float32)],
    )(xt, xh)

    idx_flat = knn.reshape(_BC * _N * _K)

    mesh = plsc.VectorSubcoreMesh(core_axis_name="c", subcore_axis_name="s")
    pooled = pl.kernel(
        _sc_pool_body,
        mesh=mesh,
        out_type=jax.ShapeDtypeStruct((_BC * _N, _FP), jnp.float32),
        scratch_types=[
            pltpu.VMEM((_G * _K,), jnp.int32),
            pltpu.VMEM((_G * _K,), jnp.int32),
            pltpu.VMEM((_G * _K, _FP), jnp.float32),
            pltpu.VMEM((_G * _K, _FP), jnp.float32),
            pltpu.VMEM((_G, _FP), jnp.float32),
            pltpu.SemaphoreType.DMA,
            pltpu.SemaphoreType.DMA,
        ],
    )(res, idx_flat)
    return pooled


def kernel(x, W1, b1, W2, b2):
    w1t = W1.T
    w2t = jnp.pad(W2, ((0, _FP - _F), (0, 0))).T
    b1r = b1.reshape(1, _F)
    b2r = jnp.pad(b2, (0, _FP - _F)).reshape(1, _FP)

    pooled = [_chunk_pipeline(x[c:c + _BC], w1t, b1r, w2t, b2r)
              for c in range(0, _B, _BC)]

    pooled3 = jnp.concatenate(pooled).reshape(_B, _N, _FP)
    out = pl.pallas_call(
        _finalize_body,
        grid=(_B,),
        in_specs=[
            pl.BlockSpec((1, _N, _FP), lambda b: (b, 0, 0)),
            pl.BlockSpec((1, _N, _IN), lambda b: (b, 0, 0)),
        ],
        out_specs=pl.BlockSpec((1, _N, 2 * _F + _IN), lambda b: (b, 0, 0)),
        out_shape=jax.ShapeDtypeStruct((_B, _N, 2 * _F + _IN), jnp.float32),
    )(pooled3, x)
    return out


# speedup vs baseline: 1.5162x; 1.0154x over previous
import functools
import jax
import jax.numpy as jnp
from jax import lax
from jax.experimental import pallas as pl
from jax.experimental.pallas import tpu as pltpu
from jax.experimental.pallas import tpu_sc as plsc

_B, _N, _IN = 8, 2048, 3
_K = 16
_F = 254
_FP = 256
_R = 512
_NW = 32
_BC = 1
_PPW = (_BC * _N) // _NW
_G = 8
_GROUPS = _PPW // _G


def _mlp_body(x_ref, w1_ref, b1_ref, w2_ref, b2_ref, o_ref):
    xb = x_ref[...]
    h = jnp.dot(xb, w1_ref[...], preferred_element_type=jnp.float32)
    h = h + b1_ref[...]
    h = 0.5 * h * (1.0 + lax.erf(h * 0.7071067811865476))
    r = jnp.dot(h, w2_ref[...], preferred_element_type=jnp.float32)
    o_ref[...] = r + b2_ref[...]


def _topk_body(xt_ref, x_ref, o_ref, dist_ref):
    b = pl.program_id(0)
    xa = xt_ref[0]
    xb = x_ref[0]
    dots = jnp.dot(xb, xa, preferred_element_type=jnp.float32)
    sq_a = jnp.sum(xa * xa, axis=0, keepdims=True)
    sq_b = jnp.sum(xb * xb, axis=1, keepdims=True)
    dist_ref[...] = sq_b + sq_a - 2.0 * dots
    iota_f = lax.broadcasted_iota(jnp.int32, (_R, _N), 1).astype(jnp.float32)
    base = b * _N
    for k in range(_K):
        d = dist_ref[...]
        m = jnp.min(d, axis=1, keepdims=True)
        eq = d == m
        idxf = jnp.min(jnp.where(eq, iota_f, jnp.float32(2 * _N)), axis=1)
        o_ref[0, :, k] = idxf.astype(jnp.int32) + base
        dist_ref[...] = jnp.where(iota_f == idxf[:, None], jnp.inf, d)


def _sc_pool_body(res_hbm, idx_hbm, out_hbm, idx0_v, idx1_v, rows0_v, rows1_v,
                  outb_v, sem0, sem1):
    wid = lax.axis_index("s") * 2 + lax.axis_index("c")
    base_pt = wid * _PPW

    def start(g, idxb, rowsb, sem):
        pltpu.sync_copy(
            idx_hbm.at[pl.ds((base_pt + g * _G) * _K, _G * _K)], idxb)
        pltpu.async_copy(res_hbm.at[idxb], rowsb, sem)

    def wait(rowsb, sem):
        pltpu.make_async_copy(res_hbm.at[pl.ds(0, _G * _K)], rowsb, sem).wait()

    def compute(g, rowsb):
        def point(pt, _):
            r0 = pt * _K
            for fc in range(_FP // 16):
                v = [rowsb[r0 + nb, pl.ds(fc * 16, 16)] for nb in range(_K)]
                while len(v) > 1:
                    v = [jnp.maximum(v[2 * i], v[2 * i + 1])
                         for i in range(len(v) // 2)]
                outb_v[pt, pl.ds(fc * 16, 16)] = v[0]
            return 0

        lax.fori_loop(0, _G, point, 0)
        pltpu.sync_copy(outb_v, out_hbm.at[pl.ds(base_pt + g * _G, _G)])

    start(0, idx0_v, rows0_v, sem0)

    def pair(p, _):
        g0 = 2 * p
        start(g0 + 1, idx1_v, rows1_v, sem1)
        wait(rows0_v, sem0)
        compute(g0, rows0_v)
        start(jnp.minimum(g0 + 2, _GROUPS - 1), idx0_v, rows0_v, sem0)
        wait(rows1_v, sem1)
        compute(g0 + 1, rows1_v)
        return 0

    lax.fori_loop(0, _GROUPS // 2, pair, 0)
    wait(rows0_v, sem0)


def _finalize_body(p_ref, x_ref, o_ref):
    pm = p_ref[0][:, :_F]
    gv = jnp.max(pm, axis=0, keepdims=True)
    o_ref[0] = jnp.concatenate(
        [pm, jnp.broadcast_to(gv, (_N, _F)), x_ref[0]], axis=1)


def _chunk_pipeline(xh, w1t, b1r, w2t, b2r):
    x2d = xh.reshape(_BC * _N, _IN)
    res = pl.pallas_call(
        _mlp_body,
        grid=(_BC * _N // 512,),
        in_specs=[
            pl.BlockSpec((512, _IN), lambda i: (i, 0)),
            pl.BlockSpec((_IN, _F), lambda i: (0, 0)),
            pl.BlockSpec((1, _F), lambda i: (0, 0)),
            pl.BlockSpec((_F, _FP), lambda i: (0, 0)),
            pl.BlockSpec((1, _FP), lambda i: (0, 0)),
        ],
        out_specs=pl.BlockSpec((512, _FP), lambda i: (i, 0)),
        out_shape=jax.ShapeDtypeStruct((_BC * _N, _FP), jnp.float32),
    )(x2d, w1t, b1r, w2t, b2r)

    xt = jnp.transpose(xh, (0, 2, 1))
    knn = pl.pallas_call(
        _topk_body,
        grid=(_BC, _N // _R),
        in_specs=[
            pl.BlockSpec((1, _IN, _N), lambda b, i: (b, 0, 0)),
            pl.BlockSpec((1, _R, _IN), lambda b, i: (b, i, 0)),
        ],
        out_specs=pl.BlockSpec((1, _R, _K), lambda b, i: (b, i, 0)),
        out_shape=jax.ShapeDtypeStruct((_BC, _N, _K), jnp.int32),
        scratch_shapes=[pltpu.VMEM((_R, _N), jnp.float32)],
    )(xt, xh)

    idx_flat = knn.reshape(_BC * _N * _K)

    mesh = plsc.VectorSubcoreMesh(core_axis_name="c", subcore_axis_name="s")
    pooled = pl.kernel(
        _sc_pool_body,
        mesh=mesh,
        out_type=jax.ShapeDtypeStruct((_BC * _N, _FP), jnp.float32),
        scratch_types=[
            pltpu.VMEM((_G * _K,), jnp.int32),
            pltpu.VMEM((_G * _K,), jnp.int32),
            pltpu.VMEM((_G * _K, _FP), jnp.float32),
            pltpu.VMEM((_G * _K, _FP), jnp.float32),
            pltpu.VMEM((_G, _FP), jnp.float32),
            pltpu.SemaphoreType.DMA,
            pltpu.SemaphoreType.DMA,
        ],
    )(res, idx_flat)
    return pooled


def kernel(x, W1, b1, W2, b2):
    w1t = W1.T
    w2t = jnp.pad(W2, ((0, _FP - _F), (0, 0))).T
    b1r = b1.reshape(1, _F)
    b2r = jnp.pad(b2, (0, _FP - _F)).reshape(1, _FP)

    pooled = [_chunk_pipeline(x[c:c + _BC], w1t, b1r, w2t, b2r)
              for c in range(0, _B, _BC)]

    pooled3 = jnp.concatenate(pooled).reshape(_B, _N, _FP)
    out = pl.pallas_call(
        _finalize_body,
        grid=(_B,),
        in_specs=[
            pl.BlockSpec((1, _N, _FP), lambda b: (b, 0, 0)),
            pl.BlockSpec((1, _N, _IN), lambda b: (b, 0, 0)),
        ],
        out_specs=pl.BlockSpec((1, _N, 2 * _F + _IN), lambda b: (b, 0, 0)),
        out_shape=jax.ShapeDtypeStruct((_B, _N, 2 * _F + _IN), jnp.float32),
    )(pooled3, x)
    return out
